# Initial kernel scaffold; baseline (speedup 1.0000x reference)
import jax
import jax.numpy as jnp
from jax.experimental import pallas as pl


def kernel(x, level_table, channel_weight, am_weight):
    raise NotImplementedError("write your pallas kernel here")


# speedup vs baseline: 8.7370x; 8.7370x over previous
import jax
import jax.numpy as jnp
from jax import lax
from jax.experimental import pallas as pl
from jax.experimental.pallas import tpu as pltpu

_N = 4
_LOW, _HIGH = 0.0, 20.0
_B, _T, _C, _D, _L, _K = 32, 128, 4, 4096, 64, 5


def _body(x_ref, lvl_ref, ch_ref, am_ref, out_ref):
    xb = x_ref[0]
    idx = jnp.clip(
        jnp.round((xb - _LOW) / (_HIGH - _LOW) * (_L - 1)), 0, _L - 1
    ).astype(jnp.int32)

    lvl = lvl_ref[:]
    ch = ch_ref[:]
    bound = (ch[:, None, :] * lvl[None, :, :]).reshape(_C * _L, _D)

    col = lax.broadcasted_iota(jnp.int32, (_T, _C * _L), 1)
    idxe = jnp.concatenate(
        [jnp.broadcast_to(idx[:, c : c + 1], (_T, _L)) for c in range(_C)], axis=1
    )
    onehot = (idxe == col % _L).astype(jnp.bfloat16)

    s = lax.dot_general(
        onehot, bound.astype(jnp.bfloat16),
        (((1,), (0,)), ((), ())),
        preferred_element_type=jnp.float32,
    )

    r3 = pltpu.roll(s, 3, axis=1)
    r2 = pltpu.roll(s, 2, axis=1)
    r1 = pltpu.roll(s, 1, axis=1)
    prod = (r3[0 : _T - 3] * r2[1 : _T - 2]) * (r1[2 : _T - 1] * s[3:_T])
    enc = jnp.sum(prod, axis=0, keepdims=True)

    am = am_ref[:]
    enc_norm = jnp.sqrt(jnp.sum(enc * enc)) + 1e-12
    am_norm = jnp.sqrt(jnp.sum(am * am, axis=1, keepdims=True)) + 1e-12
    dots = lax.dot_general(
        enc, am, (((1,), (1,)), ((), ())), preferred_element_type=jnp.float32
    )
    out_ref[...] = (dots / (enc_norm * am_norm.T))[None]


def kernel(x, level_table, channel_weight, am_weight):
    return pl.pallas_call(
        _body,
        grid=(_B,),
        in_specs=[
            pl.BlockSpec((1, _T, _C), lambda b: (b, 0, 0)),
            pl.BlockSpec((_L, _D), lambda b: (0, 0)),
            pl.BlockSpec((_C, _D), lambda b: (0, 0)),
            pl.BlockSpec((_K, _D), lambda b: (0, 0)),
        ],
        out_specs=pl.BlockSpec((1, 1, _K), lambda b: (b, 0, 0)),
        out_shape=jax.ShapeDtypeStruct((_B, 1, _K), jnp.float32),
    )(x, level_table, channel_weight, am_weight).reshape(_B, _K)
